# baseline (speedup 1.0000x reference)
import jax
import jax.numpy as jnp
from jax import lax
from jax.experimental import pallas as pl
from jax.experimental.pallas import tpu as pltpu
from jax.experimental.pallas import tpu_sc as plsc

VOCAB = 1000000
D = 64
SEQ = 200
BATCH = 4096
OUT = 2
L = 16
NC, NS = 2, 16
NW = NC * NS
BPW = BATCH // NW
HALF = SEQ // 2
NCHUNK = D // L


def _sc_body(idx_hbm, table_hbm, params_hbm, out_hbm,
             idx_v, buf_a, buf_b, params_v, out_v, sem_a, sem_b):
    wid = lax.axis_index("s") * NC + lax.axis_index("c")
    ibase = wid * (2 * BPW)
    obase = wid * BPW

    pltpu.sync_copy(params_hbm, params_v)
    pltpu.sync_copy(idx_hbm.at[pl.ds(ibase, 2 * BPW)], idx_v)

    w0 = [params_v[0, pl.ds(k * L, L)] for k in range(NCHUNK)]
    w1 = [params_v[1, pl.ds(k * L, L)] for k in range(NCHUNK)]
    b0 = params_v[0, pl.ds(D, L)][0]
    b1 = params_v[1, pl.ds(D, L)][0]
    lanes = lax.iota(jnp.int32, L)
    cols = jnp.minimum(lanes, 1)
    out_mask = lanes < OUT

    def start(buf, sem, lb):
        pltpu.make_async_copy(table_hbm.at[idx_v.at[2 * lb]],
                              buf.at[pl.ds(0, HALF)], sem).start()
        pltpu.make_async_copy(table_hbm.at[idx_v.at[2 * lb + 1]],
                              buf.at[pl.ds(HALF, HALF)], sem).start()

    def wait(buf, sem):
        pltpu.make_async_copy(table_hbm.at[idx_v.at[0]],
                              buf.at[pl.ds(0, HALF)], sem).wait()
        pltpu.make_async_copy(table_hbm.at[idx_v.at[0]],
                              buf.at[pl.ds(HALF, HALF)], sem).wait()

    def process(buf, lb):
        def row_body(i, acc):
            a0, a1, a2, a3 = acc
            for u in range(8):
                j = i * 8 + u
                a0 = a0 + buf[j, pl.ds(0 * L, L)]
                a1 = a1 + buf[j, pl.ds(1 * L, L)]
                a2 = a2 + buf[j, pl.ds(2 * L, L)]
                a3 = a3 + buf[j, pl.ds(3 * L, L)]
            return a0, a1, a2, a3

        z = jnp.zeros((L,), jnp.float32)
        a0, a1, a2, a3 = lax.fori_loop(0, SEQ // 8, row_body, (z, z, z, z))
        y0 = jnp.sum(a0 * w0[0] + a1 * w0[1] + a2 * w0[2] + a3 * w0[3]) + b0
        y1 = jnp.sum(a0 * w1[0] + a1 * w1[1] + a2 * w1[2] + a3 * w1[3]) + b1
        vals = jnp.where(lanes == 0, jnp.broadcast_to(y0, (L,)),
                         jnp.broadcast_to(y1, (L,)))
        rows = jnp.broadcast_to(lb, (L,)).astype(jnp.int32)
        plsc.store_scatter(out_v, [rows, cols], vals, mask=out_mask)

    start(buf_a, sem_a, 0)

    def pair_body(i, carry):
        start(buf_b, sem_b, 2 * i + 1)
        wait(buf_a, sem_a)
        process(buf_a, 2 * i)

        @pl.when(i < BPW // 2 - 1)
        def _():
            start(buf_a, sem_a, 2 * i + 2)

        wait(buf_b, sem_b)
        process(buf_b, 2 * i + 1)
        return carry

    lax.fori_loop(0, BPW // 2, pair_body, 0)
    pltpu.sync_copy(out_v, out_hbm.at[pl.ds(obase, BPW)])


def kernel(text, embedding, W, b):
    idx = text.T.reshape(BATCH * 2, HALF)
    wt = (W.astype(jnp.float32) / SEQ).T
    params = jnp.zeros((2, 80), jnp.float32)
    params = params.at[:, :D].set(wt).at[:, D].set(b.astype(jnp.float32))

    mesh = plsc.VectorSubcoreMesh(core_axis_name="c", subcore_axis_name="s",
                                  num_cores=NC, num_subcores=NS)
    run = pl.kernel(
        _sc_body,
        out_type=jax.ShapeDtypeStruct((BATCH, OUT), jnp.float32),
        mesh=mesh,
        compiler_params=pltpu.CompilerParams(needs_layout_passes=False,
                                              use_tc_tiling_on_sc=False),
        scratch_types=[
            pltpu.VMEM((2 * BPW, HALF), jnp.int32),
            pltpu.VMEM((SEQ, D), jnp.float32),
            pltpu.VMEM((SEQ, D), jnp.float32),
            pltpu.VMEM((2, 80), jnp.float32),
            pltpu.VMEM((BPW, OUT), jnp.float32),
            pltpu.SemaphoreType.DMA,
            pltpu.SemaphoreType.DMA,
        ],
    )
    return run(idx, embedding, params)
